# baseline (speedup 1.0000x reference)
import functools

import jax
import jax.numpy as jnp
import numpy as np
from jax import lax
from jax.experimental import pallas as pl
from jax.experimental.pallas import tpu as pltpu
from jax.experimental.pallas import tpu_sc as plsc

B = 4096
NUM_F = 26
V = 100000
D = 32
D_MODEL = 512
T = 8
CHUNK = 104
SINGLE = 20
MULTI = 6
HIST = 20
TOTAL_INT = SINGLE + MULTI * HIST
ROWS_PER_TABLE = V + 1

NW = 32
N_SUB = 8
IDX_PER_SUB = N_SUB * TOTAL_INT
IDX_ROWS = (IDX_PER_SUB + 127) // 128
SUBS_TOTAL = B // N_SUB
SUBS_PER_TILE = SUBS_TOTAL // NW

_FID = np.concatenate(
    [np.arange(SINGLE), np.repeat(np.arange(SINGLE, NUM_F), HIST)]
)
_OFFS = np.asarray(_FID * ROWS_PER_TABLE, dtype=np.int32)


N_GRP = N_SUB * MULTI


def _sc_body(table_ref, gidx_ref, midx_ref, outs_ref, outm_ref, idxbuf, gath,
             mbuf, mout, sem):
    c = lax.axis_index("c")
    s = lax.axis_index("s")
    wid = s * 2 + c

    def sub_body(i, carry):
        cid = wid * SUBS_PER_TILE + i
        pltpu.sync_copy(gidx_ref.at[cid], idxbuf)
        pltpu.sync_copy(midx_ref.at[cid], mbuf)
        copies = [
            pltpu.async_copy(
                table_ref.at[idxbuf.at[j]],
                gath.at[pl.ds(j * 128, 128)],
                sem,
            )
            for j in range(IDX_ROWS)
        ]
        svecs = []
        for h in range(N_GRP // 16):
            acc = jnp.where(mbuf[0, pl.ds(h * 16, 16)] != 0, 1.0, 0.0)
            for k in range(1, HIST):
                acc = acc + jnp.where(
                    mbuf[k, pl.ds(h * 16, 16)] != 0, 1.0, 0.0
                )
            svecs.append(1.0 / jnp.maximum(acc, 1.0))
        for cp in copies:
            cp.wait()
        for r in range(N_SUB):
            base = r * TOTAL_INT
            pltpu.sync_copy(
                gath.at[pl.ds(base, SINGLE)],
                outs_ref.at[cid * N_SUB + r],
            )
            for m in range(MULTI):
                g0 = base + SINGLE + m * HIST
                a0 = gath[g0, pl.ds(0, 16)]
                a1 = gath[g0, pl.ds(16, 16)]
                for j2 in range(1, HIST):
                    a0 = a0 + gath[g0 + j2, pl.ds(0, 16)]
                    a1 = a1 + gath[g0 + j2, pl.ds(16, 16)]
                g = r * MULTI + m
                scale = svecs[g // 16][g % 16]
                mout[g, pl.ds(0, 16)] = a0 * scale
                mout[g, pl.ds(16, 16)] = a1 * scale
        pltpu.sync_copy(mout, outm_ref.at[cid])
        return carry

    lax.fori_loop(0, SUBS_PER_TILE, sub_body, 0)


def _tc_body(cs_ref, cm_ref, w_ref, b_ref, g_ref, bt_ref, o_ref):
    x = jnp.concatenate([cs_ref[...], cm_ref[...]], axis=-1)
    for t in range(T):
        h = jnp.dot(
            x[:, t * CHUNK:(t + 1) * CHUNK],
            w_ref[t],
            preferred_element_type=jnp.float32,
        ) + b_ref[t][None, :]
        mu = jnp.mean(h, axis=-1, keepdims=True)
        var = jnp.mean((h - mu) ** 2, axis=-1, keepdims=True)
        y = (h - mu) * lax.rsqrt(var + 1e-5) * g_ref[t][None, :] + bt_ref[t][None, :]
        o_ref[:, t, :] = y * (1.0 / (1.0 + jnp.exp(-y)))


_BM = 256


@jax.jit
def kernel(int_feats, tables, W, b, gamma, beta):
    table_flat = tables.reshape(NUM_F * ROWS_PER_TABLE, D)
    gidx = int_feats + _OFFS[None, :]
    gidx = jnp.pad(
        gidx.reshape(SUBS_TOTAL, IDX_PER_SUB),
        ((0, 0), (0, IDX_ROWS * 128 - IDX_PER_SUB)),
    ).reshape(SUBS_TOTAL, IDX_ROWS, 128)
    midx = (
        int_feats[:, SINGLE:]
        .reshape(SUBS_TOTAL, N_SUB, MULTI, HIST)
        .transpose(0, 3, 1, 2)
        .reshape(SUBS_TOTAL, HIST, N_GRP)
    )

    sc_fn = pl.kernel(
        _sc_body,
        out_type=[
            jax.ShapeDtypeStruct((B, SINGLE, D), jnp.float32),
            jax.ShapeDtypeStruct((SUBS_TOTAL, N_GRP, D), jnp.float32),
        ],
        mesh=plsc.VectorSubcoreMesh(core_axis_name="c", subcore_axis_name="s"),
        compiler_params=pltpu.CompilerParams(use_tc_tiling_on_sc=False),
        scratch_types=[
            pltpu.VMEM((IDX_ROWS, 128), jnp.int32),
            pltpu.VMEM((IDX_ROWS * 128, D), jnp.float32),
            pltpu.VMEM((HIST, N_GRP), jnp.int32),
            pltpu.VMEM((N_GRP, D), jnp.float32),
            pltpu.SemaphoreType.DMA,
        ],
    )
    cat_s, cat_m = sc_fn(table_flat, gidx, midx)
    cat_s = cat_s.reshape(B, SINGLE * D)
    cat_m = cat_m.reshape(B, MULTI * D)

    out = pl.pallas_call(
        _tc_body,
        grid=(B // _BM,),
        in_specs=[
            pl.BlockSpec((_BM, SINGLE * D), lambda i: (i, 0)),
            pl.BlockSpec((_BM, MULTI * D), lambda i: (i, 0)),
            pl.BlockSpec((T, CHUNK, D_MODEL), lambda i: (0, 0, 0)),
            pl.BlockSpec((T, D_MODEL), lambda i: (0, 0)),
            pl.BlockSpec((T, D_MODEL), lambda i: (0, 0)),
            pl.BlockSpec((T, D_MODEL), lambda i: (0, 0)),
        ],
        out_specs=pl.BlockSpec((_BM, T, D_MODEL), lambda i: (i, 0, 0)),
        out_shape=jax.ShapeDtypeStruct((B, T, D_MODEL), jnp.float32),
    )(cat_s, cat_m, W, b, gamma, beta)
    return out


# speedup vs baseline: 10.6386x; 10.6386x over previous
import functools

import jax
import jax.numpy as jnp
import numpy as np
from jax import lax
from jax.experimental import pallas as pl
from jax.experimental.pallas import tpu as pltpu
from jax.experimental.pallas import tpu_sc as plsc

B = 4096
NUM_F = 26
V = 100000
D = 32
D_MODEL = 512
T = 8
CHUNK = 104
SINGLE = 20
MULTI = 6
HIST = 20
TOTAL_INT = SINGLE + MULTI * HIST
ROWS_PER_TABLE = V + 1

NW = 32
N_SUB = 8
IDX_PER_SUB = N_SUB * TOTAL_INT
IDX_ROWS = (IDX_PER_SUB + 127) // 128
SUBS_TOTAL = B // N_SUB
SUBS_PER_TILE = SUBS_TOTAL // NW

_FID = np.concatenate(
    [np.arange(SINGLE), np.repeat(np.arange(SINGLE, NUM_F), HIST)]
)
PR = 25600
PV = 4 * PR
TBL_ROWS = NUM_F * PR
_OFFS_Q = np.asarray(_FID * PV, dtype=np.int32)


N_GRP = N_SUB * MULTI


def _repack_body(x0, x1, x2, x3, x4, o_ref):
    p3 = jnp.where(pl.program_id(1) < 22, x3[0], x4[0])
    o_ref[...] = jnp.concatenate([x0[0].T, x1[0].T, x2[0].T, p3.T], axis=1)


def _sc_body(table_ref, gidx_ref, midx_ref, outs_ref, outm_ref, idxbuf, gath,
             mbuf, mout, sem):
    c = lax.axis_index("c")
    s = lax.axis_index("s")
    wid = s * 2 + c

    def sub_body(i, carry):
        cid = wid * SUBS_PER_TILE + i
        pltpu.sync_copy(gidx_ref.at[cid], idxbuf)
        pltpu.sync_copy(midx_ref.at[cid], mbuf)
        copies = [
            pltpu.async_copy(
                table_ref.at[idxbuf.at[j]],
                gath.at[pl.ds(j * 128, 128)],
                sem,
            )
            for j in range(IDX_ROWS)
        ]
        svecs = []
        for h in range(N_GRP // 16):
            acc = jnp.where(mbuf[0, pl.ds(h * 16, 16)] != 0, 1.0, 0.0)
            for k in range(1, HIST):
                acc = acc + jnp.where(
                    mbuf[k, pl.ds(h * 16, 16)] != 0, 1.0, 0.0
                )
            svecs.append(1.0 / jnp.maximum(acc, 1.0))
        for cp in copies:
            cp.wait()
        for r in range(N_SUB):
            base = r * TOTAL_INT
            pltpu.sync_copy(
                gath.at[pl.ds(base, SINGLE)],
                outs_ref.at[cid * N_SUB + r],
            )
            for m in range(MULTI):
                g0 = base + SINGLE + m * HIST
                a0 = gath[g0, pl.ds(0, 16)]
                a1 = gath[g0, pl.ds(16, 16)]
                for j2 in range(1, HIST):
                    a0 = a0 + gath[g0 + j2, pl.ds(0, 16)]
                    a1 = a1 + gath[g0 + j2, pl.ds(16, 16)]
                g = r * MULTI + m
                scale = svecs[g // 16][g % 16]
                mout[g, pl.ds(0, 16)] = a0 * scale
                mout[g, pl.ds(16, 16)] = a1 * scale
        pltpu.sync_copy(mout, outm_ref.at[cid])
        return carry

    lax.fori_loop(0, SUBS_PER_TILE, sub_body, 0)


def _tc_body(cs_ref, cm_ref, w_ref, b_ref, g_ref, bt_ref, o_ref):
    x = jnp.concatenate([cs_ref[...], cm_ref[...]], axis=-1)
    for t in range(T):
        h = jnp.dot(
            x[:, t * CHUNK:(t + 1) * CHUNK],
            w_ref[t],
            preferred_element_type=jnp.float32,
        ) + b_ref[t][None, :]
        mu = jnp.mean(h, axis=-1, keepdims=True)
        var = jnp.mean((h - mu) ** 2, axis=-1, keepdims=True)
        y = (h - mu) * lax.rsqrt(var + 1e-5) * g_ref[t][None, :] + bt_ref[t][None, :]
        o_ref[:, t, :] = y * (1.0 / (1.0 + jnp.exp(-y)))


_BM = 256


@jax.jit
def kernel(int_feats, tables, W, b, gamma, beta):
    t2 = tables.transpose(0, 2, 1)
    tail_start = 3 * PR + 22 * 1024
    tail = jnp.pad(
        t2[:, :, tail_start:],
        ((0, 0), (0, 0), (0, 2 * 1024 - (V + 1 - tail_start))),
    )
    nvb = PR // 1024
    in_specs = [
        pl.BlockSpec((1, D, 1024), lambda f, v, bb=b_: (f, 0, bb * nvb + v))
        for b_ in range(3)
    ]
    in_specs.append(
        pl.BlockSpec(
            (1, D, 1024),
            lambda f, v: (f, 0, 3 * nvb + jnp.minimum(v, 21)),
        )
    )
    in_specs.append(
        pl.BlockSpec(
            (1, D, 1024),
            lambda f, v: (f, 0, jnp.clip(v - 22, 0, 1)),
        )
    )
    table2d = pl.pallas_call(
        _repack_body,
        grid=(NUM_F, nvb),
        in_specs=in_specs,
        out_specs=pl.BlockSpec((1024, 128), lambda f, v: (f * nvb + v, 0)),
        out_shape=jax.ShapeDtypeStruct((TBL_ROWS, 128), jnp.float32),
    )(t2, t2, t2, t2, tail)
    table_flat = table2d.reshape(4 * TBL_ROWS, D)
    gidx = _OFFS_Q[None, :] + 4 * (int_feats % PR) + (int_feats // PR)
    gidx = jnp.pad(
        gidx.reshape(SUBS_TOTAL, IDX_PER_SUB),
        ((0, 0), (0, IDX_ROWS * 128 - IDX_PER_SUB)),
    ).reshape(SUBS_TOTAL, IDX_ROWS, 128)
    midx = (
        int_feats[:, SINGLE:]
        .reshape(SUBS_TOTAL, N_SUB, MULTI, HIST)
        .transpose(0, 3, 1, 2)
        .reshape(SUBS_TOTAL, HIST, N_GRP)
    )

    sc_fn = pl.kernel(
        _sc_body,
        out_type=[
            jax.ShapeDtypeStruct((B, SINGLE, D), jnp.float32),
            jax.ShapeDtypeStruct((SUBS_TOTAL, N_GRP, D), jnp.float32),
        ],
        mesh=plsc.VectorSubcoreMesh(core_axis_name="c", subcore_axis_name="s"),
        compiler_params=pltpu.CompilerParams(use_tc_tiling_on_sc=False),
        scratch_types=[
            pltpu.VMEM((IDX_ROWS, 128), jnp.int32),
            pltpu.VMEM((IDX_ROWS * 128, D), jnp.float32),
            pltpu.VMEM((HIST, N_GRP), jnp.int32),
            pltpu.VMEM((N_GRP, D), jnp.float32),
            pltpu.SemaphoreType.DMA,
        ],
    )
    cat_s, cat_m = sc_fn(table_flat, gidx, midx)
    cat_s = cat_s.reshape(B, SINGLE * D)
    cat_m = cat_m.reshape(B, MULTI * D)

    out = pl.pallas_call(
        _tc_body,
        grid=(B // _BM,),
        in_specs=[
            pl.BlockSpec((_BM, SINGLE * D), lambda i: (i, 0)),
            pl.BlockSpec((_BM, MULTI * D), lambda i: (i, 0)),
            pl.BlockSpec((T, CHUNK, D_MODEL), lambda i: (0, 0, 0)),
            pl.BlockSpec((T, D_MODEL), lambda i: (0, 0)),
            pl.BlockSpec((T, D_MODEL), lambda i: (0, 0)),
            pl.BlockSpec((T, D_MODEL), lambda i: (0, 0)),
        ],
        out_specs=pl.BlockSpec((_BM, T, D_MODEL), lambda i: (i, 0, 0)),
        out_shape=jax.ShapeDtypeStruct((B, T, D_MODEL), jnp.float32),
    )(cat_s, cat_m, W, b, gamma, beta)
    return out
